# Initial kernel scaffold; baseline (speedup 1.0000x reference)
import jax
import jax.numpy as jnp
from jax.experimental import pallas as pl


def kernel(x, edge_index, W1, b1, W_mu, b_mu, W_lv, b_lv):
    raise NotImplementedError("write your pallas kernel here")


# speedup vs baseline: 13.9477x; 13.9477x over previous
import functools

import jax
import jax.numpy as jnp
from jax import lax
from jax.experimental import pallas as pl
from jax.experimental.pallas import tpu as pltpu
from jax.experimental.pallas import tpu_sc as plsc

N = 10000
E = 320000
D = 128
DH = 64
DO = 64
NC = 2
NS = 16
NW = NC * NS
K = 80
EPT = E // NS
EPW = E // NW
NP = 10240
RB = NP // NS

_mesh = plsc.VectorSubcoreMesh(
    core_axis_name="c", subcore_axis_name="s", num_cores=NC, num_subcores=NS
)


@functools.partial(
    pl.kernel,
    out_type=jax.ShapeDtypeStruct((NC, NP), jnp.float32),
    mesh=_mesh,
    scratch_types=[
        pltpu.VMEM((K,), jnp.int32),
        pltpu.VMEM((K,), jnp.float32),
        pltpu.VMEM((RB,), jnp.float32),
        pltpu.VMEM_SHARED((NP,), jnp.float32),
    ],
    compiler_params=pltpu.CompilerParams(use_tc_tiling_on_sc=False),
)
def _sc_degree(dst_hbm, zvec_hbm, out_hbm, idx_v, ones_v, rb_v, deg_sh):
    cid = lax.axis_index("c")
    sid = lax.axis_index("s")
    wid = sid * NC + cid
    zb = pl.multiple_of(sid * RB, 8)

    for i in range(K // 16):
        ones_v[pl.ds(16 * i, 16)] = jnp.full((16,), 1.0, jnp.float32)

    pltpu.sync_copy(zvec_hbm.at[pl.ds(zb, RB)], rb_v)
    pltpu.sync_copy(rb_v, deg_sh.at[pl.ds(zb, RB)])
    plsc.subcore_barrier()

    ebase = pl.multiple_of(wid * EPW, 8)

    def body(i, carry):
        b = pl.multiple_of(ebase + i * K, 8)
        pltpu.sync_copy(dst_hbm.at[pl.ds(b, K)], idx_v)
        pltpu.sync_copy(ones_v, deg_sh.at[idx_v], add=True)
        return carry

    lax.fori_loop(0, EPW // K, body, 0)
    plsc.subcore_barrier()

    pltpu.sync_copy(deg_sh.at[pl.ds(zb, RB)], rb_v)
    pltpu.sync_copy(rb_v, out_hbm.at[cid, pl.ds(zb, RB)])


@functools.partial(
    pl.kernel,
    out_type=(
        jax.ShapeDtypeStruct((NC, NP, DH), jnp.float32),
        jax.ShapeDtypeStruct((NC, NP), jnp.float32),
    ),
    mesh=_mesh,
    scratch_types=[
        pltpu.VMEM((K,), jnp.int32),
        pltpu.VMEM((K,), jnp.int32),
        pltpu.VMEM((K,), jnp.int32),
        pltpu.VMEM((K, DH), jnp.float32),
        pltpu.VMEM((K,), jnp.float32),
        pltpu.VMEM((RB, DH), jnp.float32),
        pltpu.VMEM((RB,), jnp.float32),
        pltpu.VMEM_SHARED((NP, DH), jnp.float32),
        pltpu.VMEM_SHARED((NP,), jnp.float32),
        pltpu.SemaphoreType.DMA,
        pltpu.SemaphoreType.DMA,
    ],
    compiler_params=pltpu.CompilerParams(use_tc_tiling_on_sc=False),
)
def _sc_edges(
    src_hbm, dst_hbm, sflat_hbm, dis_hbm, zrows_hbm, zvec_hbm,
    acc_out, csum_out,
    srci_v, gidx_v, dsti_v, rows_v, disv_v, rbr_v, rbv_v, acc_sh, csum_sh,
    sem, sem2,
):
    cid = lax.axis_index("c")
    sid = lax.axis_index("s")
    wid = sid * NC + cid
    zb = pl.multiple_of(sid * RB, 8)
    goff = cid * N

    pltpu.sync_copy(zrows_hbm.at[pl.ds(zb, RB)], rbr_v)
    pltpu.sync_copy(rbr_v, acc_sh.at[pl.ds(zb, RB)])
    pltpu.sync_copy(zvec_hbm.at[pl.ds(zb, RB)], rbv_v)
    pltpu.sync_copy(rbv_v, csum_sh.at[pl.ds(zb, RB)])
    plsc.subcore_barrier()

    rbase = pl.multiple_of(sid * EPT, 8)

    def rows_body(i, carry):
        b = pl.multiple_of(rbase + i * K, 8)
        pltpu.sync_copy(src_hbm.at[pl.ds(b, K)], srci_v)
        pltpu.sync_copy(dst_hbm.at[pl.ds(b, K)], dsti_v)
        for j in range(K // 16):
            sl = pl.ds(j * 16, 16)
            gidx_v[sl] = srci_v[sl] + goff
        pltpu.async_copy(sflat_hbm.at[gidx_v], rows_v, sem).wait()
        pltpu.sync_copy(rows_v, acc_sh.at[dsti_v], add=True)
        return carry

    lax.fori_loop(0, EPT // K, rows_body, 0)

    cbase = pl.multiple_of(wid * EPW, 8)

    def csum_body(i, carry):
        b = pl.multiple_of(cbase + i * K, 8)
        pltpu.sync_copy(src_hbm.at[pl.ds(b, K)], srci_v)
        pltpu.sync_copy(dst_hbm.at[pl.ds(b, K)], dsti_v)
        pltpu.async_copy(dis_hbm.at[dsti_v], disv_v, sem2).wait()
        pltpu.sync_copy(disv_v, csum_sh.at[srci_v], add=True)
        return carry

    lax.fori_loop(0, EPW // K, csum_body, 0)
    plsc.subcore_barrier()

    pltpu.sync_copy(acc_sh.at[pl.ds(zb, RB)], rbr_v)
    pltpu.sync_copy(rbr_v, acc_out.at[cid, pl.ds(zb, RB)])
    pltpu.sync_copy(csum_sh.at[pl.ds(zb, RB)], rbv_v)
    pltpu.sync_copy(rbv_v, csum_out.at[cid, pl.ds(zb, RB)])


BLK = 1000


def _tc_scale_body(x_ref, w_ref, d0_ref, d1_ref, sp_ref, dis_ref):
    xw = jnp.dot(x_ref[...], w_ref[...], preferred_element_type=jnp.float32)
    deg = d0_ref[0] + d1_ref[0] + 1.0
    dis = lax.rsqrt(deg)
    s = xw * dis
    sp_ref[0] = s[:, :DH]
    sp_ref[1] = s[:, DH:]
    dis_ref[...] = dis


def _tc_scale(x, W1, degp):
    degp3 = degp.reshape(NC, NP, 1)
    return pl.pallas_call(
        _tc_scale_body,
        grid=(N // BLK,),
        in_specs=[
            pl.BlockSpec((BLK, D), lambda i: (i, 0)),
            pl.BlockSpec((D, D), lambda i: (0, 0)),
            pl.BlockSpec((1, BLK, 1), lambda i: (0, i, 0)),
            pl.BlockSpec((1, BLK, 1), lambda i: (1, i, 0)),
        ],
        out_specs=[
            pl.BlockSpec((NC, BLK, DH), lambda i: (0, i, 0)),
            pl.BlockSpec((BLK, 1), lambda i: (i, 0)),
        ],
        out_shape=[
            jax.ShapeDtypeStruct((NC, N, DH), jnp.float32),
            jax.ShapeDtypeStruct((N, 1), jnp.float32),
        ],
        compiler_params=pltpu.CompilerParams(
            dimension_semantics=("arbitrary",),
        ),
    )(x, W1, degp3, degp3)


def _tc_final_body(
    s0_ref, s1_ref, dis_ref, a0_ref, a1_ref, c0_ref, c1_ref, b1_ref,
    wmu_ref, bmu_ref, wlv_ref, blv_ref, mu_ref, lv_ref, g_ref,
):
    i = pl.program_id(0)

    @pl.when(i == 0)
    def _():
        g_ref[...] = jnp.zeros((1, D), jnp.float32)

    dis = dis_ref[...]
    acc = jnp.concatenate([a0_ref[0], a1_ref[0]], axis=1)
    s = jnp.concatenate([s0_ref[0], s1_ref[0]], axis=1)
    h = jnp.maximum(dis * acc + dis * s + b1_ref[...], 0.0)
    csum = c0_ref[0] + c1_ref[0]
    c = dis * (dis + csum)
    g_ref[...] += jnp.sum(c * h, axis=0, keepdims=True)

    @pl.when(i == pl.num_programs(0) - 1)
    def _():
        g = g_ref[...]
        fn = jnp.float32(N)
        mu_ref[...] = (
            jnp.dot(g, wmu_ref[...], preferred_element_type=jnp.float32)
            + fn * bmu_ref[...]
        )
        lv_ref[...] = (
            jnp.dot(g, wlv_ref[...], preferred_element_type=jnp.float32)
            + fn * blv_ref[...]
        )


def _tc_final(s_pair, dis, accp, csump, b1, W_mu, b_mu, W_lv, b_lv):
    csump3 = csump.reshape(NC, NP, 1)
    return pl.pallas_call(
        _tc_final_body,
        grid=(N // BLK,),
        in_specs=[
            pl.BlockSpec((1, BLK, DH), lambda i: (0, i, 0)),
            pl.BlockSpec((1, BLK, DH), lambda i: (1, i, 0)),
            pl.BlockSpec((BLK, 1), lambda i: (i, 0)),
            pl.BlockSpec((1, BLK, DH), lambda i: (0, i, 0)),
            pl.BlockSpec((1, BLK, DH), lambda i: (1, i, 0)),
            pl.BlockSpec((1, BLK, 1), lambda i: (0, i, 0)),
            pl.BlockSpec((1, BLK, 1), lambda i: (1, i, 0)),
            pl.BlockSpec((1, D), lambda i: (0, 0)),
            pl.BlockSpec((D, DO), lambda i: (0, 0)),
            pl.BlockSpec((1, DO), lambda i: (0, 0)),
            pl.BlockSpec((D, DO), lambda i: (0, 0)),
            pl.BlockSpec((1, DO), lambda i: (0, 0)),
        ],
        out_specs=[
            pl.BlockSpec((1, DO), lambda i: (0, 0)),
            pl.BlockSpec((1, DO), lambda i: (0, 0)),
        ],
        out_shape=[
            jax.ShapeDtypeStruct((1, DO), jnp.float32),
            jax.ShapeDtypeStruct((1, DO), jnp.float32),
        ],
        scratch_shapes=[pltpu.VMEM((1, D), jnp.float32)],
        compiler_params=pltpu.CompilerParams(
            dimension_semantics=("arbitrary",),
        ),
    )(s_pair, s_pair, dis, accp, accp, csump3, csump3, b1.reshape(1, D),
      W_mu, b_mu.reshape(1, DO), W_lv, b_lv.reshape(1, DO))


def kernel(x, edge_index, W1, b1, W_mu, b_mu, W_lv, b_lv):
    src = edge_index[0].astype(jnp.int32)
    dst = edge_index[1].astype(jnp.int32)
    zvec = jnp.zeros((NP,), jnp.float32)
    zrows = jnp.zeros((NP, DH), jnp.float32)

    degp = _sc_degree(dst, zvec)
    s_pair, dis = _tc_scale(x, W1, degp)
    accp, csump = _sc_edges(
        src, dst, s_pair.reshape(NC * N, DH), dis.reshape(N), zrows, zvec
    )
    mu, lv = _tc_final(s_pair, dis, accp, csump, b1, W_mu, b_mu, W_lv, b_lv)
    return (mu, lv)


# speedup vs baseline: 39.1915x; 2.8099x over previous
import functools

import jax
import jax.numpy as jnp
from jax import lax
from jax.experimental import pallas as pl
from jax.experimental.pallas import tpu as pltpu
from jax.experimental.pallas import tpu_sc as plsc

N = 10000
E = 320000
D = 128
DH = 64
DO = 64
NC = 2
NS = 16
NW = NC * NS
K = 80
EPT = E // NS
EPW = E // NW
NP = 10240
RB = NP // NS

_mesh = plsc.VectorSubcoreMesh(
    core_axis_name="c", subcore_axis_name="s", num_cores=NC, num_subcores=NS
)


@functools.partial(
    pl.kernel,
    out_type=jax.ShapeDtypeStruct((NC, NP), jnp.float32),
    mesh=_mesh,
    scratch_types=[
        pltpu.VMEM((K,), jnp.int32),
        pltpu.VMEM((K,), jnp.int32),
        pltpu.VMEM((K,), jnp.int32),
        pltpu.VMEM((K,), jnp.int32),
        pltpu.VMEM((K,), jnp.float32),
        pltpu.VMEM((RB,), jnp.float32),
        pltpu.VMEM_SHARED((NP,), jnp.float32),
        pltpu.SemaphoreType.DMA,
        pltpu.SemaphoreType.DMA,
        pltpu.SemaphoreType.DMA,
        pltpu.SemaphoreType.DMA,
    ],
    compiler_params=pltpu.CompilerParams(use_tc_tiling_on_sc=False),
)
def _sc_degree(
    dst_hbm, zvec_hbm, out_hbm,
    dst0, dst1, sx0, sx1, ones_v, rb_v, deg_sh, li0, li1, ls0, ls1,
):
    cid = lax.axis_index("c")
    sid = lax.axis_index("s")
    wid = sid * NC + cid
    zb = pl.multiple_of(sid * RB, 8)
    dsti = (dst0, dst1)
    sidx = (sx0, sx1)
    isem = (li0, li1)
    ssem = (ls0, ls1)

    for i in range(K // 16):
        ones_v[pl.ds(16 * i, 16)] = jnp.full((16,), 1.0, jnp.float32)

    pltpu.sync_copy(zvec_hbm.at[pl.ds(zb, RB)], rb_v)
    pltpu.sync_copy(rb_v, deg_sh.at[pl.ds(zb, RB)])
    plsc.subcore_barrier()

    ebase = pl.multiple_of(wid * EPW, 8)
    NCH = EPW // K
    NPAIR = (NCH - 1) // 2

    def coff(c):
        return pl.multiple_of(ebase + c * K, 8)

    for b in range(2):
        pltpu.async_copy(dst_hbm.at[pl.ds(coff(b), K)], dsti[b], isem[b])

    def body(i2, carry):
        for b in range(2):
            c = i2 * 2 + b
            pltpu.make_async_copy(
                dst_hbm.at[pl.ds(coff(c), K)], dsti[b], isem[b]
            ).wait()

            @pl.when(i2 > 0)
            def _():
                pltpu.make_async_copy(
                    ones_v, deg_sh.at[sidx[b]], ssem[b]
                ).wait()

            for j in range(K // 16):
                sl = pl.ds(j * 16, 16)
                sidx[b][sl] = dsti[b][sl]

            if b == 0:
                pltpu.async_copy(
                    dst_hbm.at[pl.ds(coff(c + 2), K)], dsti[b], isem[b]
                )
            else:
                @pl.when(i2 < NPAIR - 1)
                def _():
                    pltpu.async_copy(
                        dst_hbm.at[pl.ds(coff(c + 2), K)], dsti[b], isem[b]
                    )
            pltpu.async_copy(ones_v, deg_sh.at[sidx[b]], ssem[b], add=True)
        return carry

    lax.fori_loop(0, NPAIR, body, 0)
    pltpu.make_async_copy(
        dst_hbm.at[pl.ds(coff(NCH - 1), K)], dsti[0], isem[0]
    ).wait()
    pltpu.make_async_copy(ones_v, deg_sh.at[sidx[0]], ssem[0]).wait()
    for j in range(K // 16):
        sl = pl.ds(j * 16, 16)
        sx0[sl] = dst0[sl]
    pltpu.async_copy(ones_v, deg_sh.at[sidx[0]], ssem[0], add=True)
    pltpu.make_async_copy(ones_v, deg_sh.at[sidx[0]], ssem[0]).wait()
    pltpu.make_async_copy(ones_v, deg_sh.at[sidx[1]], ssem[1]).wait()
    plsc.subcore_barrier()

    pltpu.sync_copy(deg_sh.at[pl.ds(zb, RB)], rb_v)
    pltpu.sync_copy(rb_v, out_hbm.at[cid, pl.ds(zb, RB)])


@functools.partial(
    pl.kernel,
    out_type=(
        jax.ShapeDtypeStruct((NC, NP, DH), jnp.float32),
        jax.ShapeDtypeStruct((NC, NP), jnp.float32),
    ),
    mesh=_mesh,
    scratch_types=(
        [pltpu.VMEM((K,), jnp.int32)] * 2
        + [pltpu.VMEM((K,), jnp.int32)] * 2
        + [pltpu.VMEM((K,), jnp.int32)] * 2
        + [pltpu.VMEM((K,), jnp.int32)] * 2
        + [pltpu.VMEM((K,), jnp.int32)] * 2
        + [pltpu.VMEM((K, DH), jnp.float32)] * 2
        + [pltpu.VMEM((K,), jnp.float32)] * 2
        + [
            pltpu.VMEM((RB, DH), jnp.float32),
            pltpu.VMEM((RB,), jnp.float32),
            pltpu.VMEM_SHARED((NP, DH), jnp.float32),
            pltpu.VMEM_SHARED((NP,), jnp.float32),
        ]
        + [pltpu.SemaphoreType.DMA] * 12
    ),
    compiler_params=pltpu.CompilerParams(use_tc_tiling_on_sc=False),
)
def _sc_edges(
    src_hbm, dst_hbm, sflat_hbm, dis_hbm, zrows_hbm, zvec_hbm,
    acc_out, csum_out,
    sr02, sr11, ds02, ds11, gx0, gx1, sd0, sd1, cs0, cs1, rw0, rw1, dv0, dv1,
    rbr_v, rbv_v, acc_sh, csum_sh,
    lis0, lis1, lid0, lid1, lg0, lg1, ld0, ld1, lr0, lr1, lc0, lc1,
):
    cid = lax.axis_index("c")
    sid = lax.axis_index("s")
    zb = pl.multiple_of(sid * RB, 8)
    goff = cid * N

    srci = (sr02, sr11)
    dsti = (ds02, ds11)
    gidx = (gx0, gx1)
    sdst = (sd0, sd1)
    csrc = (cs0, cs1)
    rows = (rw0, rw1)
    disv = (dv0, dv1)
    isems = (lis0, lis1)
    isemd = (lid0, lid1)
    gsem = (lg0, lg1)
    dsem = (ld0, ld1)
    ssem = (lr0, lr1)
    csem = (lc0, lc1)

    pltpu.sync_copy(zrows_hbm.at[pl.ds(zb, RB)], rbr_v)
    pltpu.sync_copy(rbr_v, acc_sh.at[pl.ds(zb, RB)])
    pltpu.sync_copy(zvec_hbm.at[pl.ds(zb, RB)], rbv_v)
    pltpu.sync_copy(rbv_v, csum_sh.at[pl.ds(zb, RB)])
    plsc.subcore_barrier()

    rbase = pl.multiple_of(sid * EPT, 8)
    NCH = EPT // K
    NPAIR = NCH // 2

    def coff(c):
        return pl.multiple_of(rbase + c * K, 8)

    for b in range(2):
        pltpu.async_copy(src_hbm.at[pl.ds(coff(b), K)], srci[b], isems[b])
        pltpu.async_copy(dst_hbm.at[pl.ds(coff(b), K)], dsti[b], isemd[b])

    def body(i2, carry):
        for b in range(2):
            c = i2 * 2 + b
            pltpu.make_async_copy(
                src_hbm.at[pl.ds(coff(c), K)], srci[b], isems[b]
            ).wait()
            pltpu.make_async_copy(
                dst_hbm.at[pl.ds(coff(c), K)], dsti[b], isemd[b]
            ).wait()

            @pl.when(i2 > 0)
            def _():
                pltpu.make_async_copy(
                    rows[b], acc_sh.at[sdst[b]], ssem[b]
                ).wait()
                pltpu.make_async_copy(
                    disv[b], csum_sh.at[csrc[b]], csem[b]
                ).wait()

            for j in range(K // 16):
                sl = pl.ds(j * 16, 16)
                sv = srci[b][sl]
                gidx[b][sl] = sv + goff
                csrc[b][sl] = sv
                sdst[b][sl] = dsti[b][sl]

            @pl.when(i2 < NPAIR - 1)
            def _():
                pltpu.async_copy(
                    src_hbm.at[pl.ds(coff(c + 2), K)], srci[b], isems[b]
                )
                pltpu.async_copy(
                    dst_hbm.at[pl.ds(coff(c + 2), K)], dsti[b], isemd[b]
                )

            pltpu.async_copy(sflat_hbm.at[gidx[b]], rows[b], gsem[b])
            pltpu.async_copy(dis_hbm.at[sdst[b]], disv[b], dsem[b])

        for b in range(2):
            pltpu.make_async_copy(sflat_hbm.at[gidx[b]], rows[b], gsem[b]).wait()
            pltpu.async_copy(rows[b], acc_sh.at[sdst[b]], ssem[b], add=True)
            pltpu.make_async_copy(dis_hbm.at[sdst[b]], disv[b], dsem[b]).wait()
            pltpu.async_copy(disv[b], csum_sh.at[csrc[b]], csem[b], add=True)
        return carry

    lax.fori_loop(0, NPAIR, body, 0)
    for b in range(2):
        pltpu.make_async_copy(rows[b], acc_sh.at[sdst[b]], ssem[b]).wait()
        pltpu.make_async_copy(disv[b], csum_sh.at[csrc[b]], csem[b]).wait()
    plsc.subcore_barrier()

    pltpu.sync_copy(acc_sh.at[pl.ds(zb, RB)], rbr_v)
    pltpu.sync_copy(rbr_v, acc_out.at[cid, pl.ds(zb, RB)])
    pltpu.sync_copy(csum_sh.at[pl.ds(zb, RB)], rbv_v)
    pltpu.sync_copy(rbv_v, csum_out.at[cid, pl.ds(zb, RB)])


BLK = 1000


def _tc_scale_body(x_ref, w_ref, d0_ref, d1_ref, sp_ref, dis_ref):
    xw = jnp.dot(x_ref[...], w_ref[...], preferred_element_type=jnp.float32)
    deg = d0_ref[0] + d1_ref[0] + 1.0
    dis = lax.rsqrt(deg)
    s = xw * dis
    sp_ref[0] = s[:, :DH]
    sp_ref[1] = s[:, DH:]
    dis_ref[...] = dis


def _tc_scale(x, W1, degp):
    degp3 = degp.reshape(NC, NP, 1)
    return pl.pallas_call(
        _tc_scale_body,
        grid=(N // BLK,),
        in_specs=[
            pl.BlockSpec((BLK, D), lambda i: (i, 0)),
            pl.BlockSpec((D, D), lambda i: (0, 0)),
            pl.BlockSpec((1, BLK, 1), lambda i: (0, i, 0)),
            pl.BlockSpec((1, BLK, 1), lambda i: (1, i, 0)),
        ],
        out_specs=[
            pl.BlockSpec((NC, BLK, DH), lambda i: (0, i, 0)),
            pl.BlockSpec((BLK, 1), lambda i: (i, 0)),
        ],
        out_shape=[
            jax.ShapeDtypeStruct((NC, N, DH), jnp.float32),
            jax.ShapeDtypeStruct((N, 1), jnp.float32),
        ],
        compiler_params=pltpu.CompilerParams(
            dimension_semantics=("arbitrary",),
        ),
    )(x, W1, degp3, degp3)


def _tc_final_body(
    s0_ref, s1_ref, dis_ref, a0_ref, a1_ref, c0_ref, c1_ref, b1_ref,
    wmu_ref, bmu_ref, wlv_ref, blv_ref, mu_ref, lv_ref, g_ref,
):
    i = pl.program_id(0)

    @pl.when(i == 0)
    def _():
        g_ref[...] = jnp.zeros((1, D), jnp.float32)

    dis = dis_ref[...]
    acc = jnp.concatenate([a0_ref[0], a1_ref[0]], axis=1)
    s = jnp.concatenate([s0_ref[0], s1_ref[0]], axis=1)
    h = jnp.maximum(dis * acc + dis * s + b1_ref[...], 0.0)
    csum = (c0_ref[0] + c1_ref[0]) * 0.5
    c = dis * (dis + csum)
    g_ref[...] += jnp.sum(c * h, axis=0, keepdims=True)

    @pl.when(i == pl.num_programs(0) - 1)
    def _():
        g = g_ref[...]
        fn = jnp.float32(N)
        mu_ref[...] = (
            jnp.dot(g, wmu_ref[...], preferred_element_type=jnp.float32)
            + fn * bmu_ref[...]
        )
        lv_ref[...] = (
            jnp.dot(g, wlv_ref[...], preferred_element_type=jnp.float32)
            + fn * blv_ref[...]
        )


def _tc_final(s_pair, dis, accp, csump, b1, W_mu, b_mu, W_lv, b_lv):
    csump3 = csump.reshape(NC, NP, 1)
    return pl.pallas_call(
        _tc_final_body,
        grid=(N // BLK,),
        in_specs=[
            pl.BlockSpec((1, BLK, DH), lambda i: (0, i, 0)),
            pl.BlockSpec((1, BLK, DH), lambda i: (1, i, 0)),
            pl.BlockSpec((BLK, 1), lambda i: (i, 0)),
            pl.BlockSpec((1, BLK, DH), lambda i: (0, i, 0)),
            pl.BlockSpec((1, BLK, DH), lambda i: (1, i, 0)),
            pl.BlockSpec((1, BLK, 1), lambda i: (0, i, 0)),
            pl.BlockSpec((1, BLK, 1), lambda i: (1, i, 0)),
            pl.BlockSpec((1, D), lambda i: (0, 0)),
            pl.BlockSpec((D, DO), lambda i: (0, 0)),
            pl.BlockSpec((1, DO), lambda i: (0, 0)),
            pl.BlockSpec((D, DO), lambda i: (0, 0)),
            pl.BlockSpec((1, DO), lambda i: (0, 0)),
        ],
        out_specs=[
            pl.BlockSpec((1, DO), lambda i: (0, 0)),
            pl.BlockSpec((1, DO), lambda i: (0, 0)),
        ],
        out_shape=[
            jax.ShapeDtypeStruct((1, DO), jnp.float32),
            jax.ShapeDtypeStruct((1, DO), jnp.float32),
        ],
        scratch_shapes=[pltpu.VMEM((1, D), jnp.float32)],
        compiler_params=pltpu.CompilerParams(
            dimension_semantics=("arbitrary",),
        ),
    )(s_pair, s_pair, dis, accp, accp, csump3, csump3, b1.reshape(1, D),
      W_mu, b_mu.reshape(1, DO), W_lv, b_lv.reshape(1, DO))


def kernel(x, edge_index, W1, b1, W_mu, b_mu, W_lv, b_lv):
    src = edge_index[0].astype(jnp.int32)
    dst = edge_index[1].astype(jnp.int32)
    zvec = jnp.zeros((NP,), jnp.float32)
    zrows = jnp.zeros((NP, DH), jnp.float32)

    degp = _sc_degree(dst, zvec)
    s_pair, dis = _tc_scale(x, W1, degp)
    accp, csump = _sc_edges(
        src, dst, s_pair.reshape(NC * N, DH), dis.reshape(N), zrows, zvec
    )
    mu, lv = _tc_final(s_pair, dis, accp, csump, b1, W_mu, b_mu, W_lv, b_lv)
    return (mu, lv)


# speedup vs baseline: 42.2306x; 1.0775x over previous
import functools

import jax
import jax.numpy as jnp
from jax import lax
from jax.experimental import pallas as pl
from jax.experimental.pallas import tpu as pltpu
from jax.experimental.pallas import tpu_sc as plsc

N = 10000
E = 320000
D = 128
DH = 64
DO = 64
NC = 2
NS = 16
NW = NC * NS
K = 80
EPT = E // NS
EPW = E // NW
NP = 10240
RB = NP // NS
L = 16

_mesh = plsc.VectorSubcoreMesh(
    core_axis_name="c", subcore_axis_name="s", num_cores=NC, num_subcores=NS
)

def _z16():
    return jnp.zeros((L,), jnp.float32)


@functools.partial(
    pl.kernel,
    out_type=jax.ShapeDtypeStruct((NC, NP), jnp.float32),
    mesh=_mesh,
    scratch_types=[
        pltpu.VMEM((K,), jnp.int32),
        pltpu.VMEM((K,), jnp.int32),
        pltpu.VMEM((K,), jnp.int32),
        pltpu.VMEM((K,), jnp.int32),
        pltpu.VMEM((K,), jnp.float32),
        pltpu.VMEM((RB,), jnp.float32),
        pltpu.VMEM_SHARED((NP,), jnp.float32),
        pltpu.SemaphoreType.DMA,
        pltpu.SemaphoreType.DMA,
        pltpu.SemaphoreType.DMA,
        pltpu.SemaphoreType.DMA,
    ],
    compiler_params=pltpu.CompilerParams(use_tc_tiling_on_sc=False),
)
def _sc_degree(
    ei_hbm, out_hbm,
    dst0, dst1, sx0, sx1, ones_v, rb_v, deg_sh, li0, li1, ls0, ls1,
):
    cid = lax.axis_index("c")
    sid = lax.axis_index("s")
    wid = sid * NC + cid
    zb = pl.multiple_of(sid * RB, 8)
    dsti = (dst0, dst1)
    sidx = (sx0, sx1)
    isem = (li0, li1)
    ssem = (ls0, ls1)

    for i in range(K // L):
        ones_v[pl.ds(L * i, L)] = jnp.full((L,), 1.0, jnp.float32)
    for i in range(RB // L):
        rb_v[pl.ds(L * i, L)] = _z16()

    pltpu.sync_copy(rb_v, deg_sh.at[pl.ds(zb, RB)])
    plsc.subcore_barrier()

    ebase = pl.multiple_of(wid * EPW, 8)
    NCH = EPW // K
    NPAIR = (NCH - 1) // 2

    def coff(c):
        return pl.multiple_of(ebase + c * K, 8)

    for b in range(2):
        pltpu.async_copy(ei_hbm.at[1, pl.ds(coff(b), K)], dsti[b], isem[b])

    def body(i2, carry):
        for b in range(2):
            c = i2 * 2 + b
            pltpu.make_async_copy(
                ei_hbm.at[1, pl.ds(coff(c), K)], dsti[b], isem[b]
            ).wait()

            @pl.when(i2 > 0)
            def _():
                pltpu.make_async_copy(
                    ones_v, deg_sh.at[sidx[b]], ssem[b]
                ).wait()

            for j in range(K // L):
                sl = pl.ds(j * L, L)
                sidx[b][sl] = dsti[b][sl]

            if b == 0:
                pltpu.async_copy(
                    ei_hbm.at[1, pl.ds(coff(c + 2), K)], dsti[b], isem[b]
                )
            else:
                @pl.when(i2 < NPAIR - 1)
                def _():
                    pltpu.async_copy(
                        ei_hbm.at[1, pl.ds(coff(c + 2), K)], dsti[b], isem[b]
                    )
            pltpu.async_copy(ones_v, deg_sh.at[sidx[b]], ssem[b], add=True)
        return carry

    lax.fori_loop(0, NPAIR, body, 0)
    pltpu.make_async_copy(
        ei_hbm.at[1, pl.ds(coff(NCH - 1), K)], dsti[0], isem[0]
    ).wait()
    pltpu.make_async_copy(ones_v, deg_sh.at[sidx[0]], ssem[0]).wait()
    for j in range(K // L):
        sl = pl.ds(j * L, L)
        sx0[sl] = dst0[sl]
    pltpu.async_copy(ones_v, deg_sh.at[sidx[0]], ssem[0], add=True)
    pltpu.make_async_copy(ones_v, deg_sh.at[sidx[0]], ssem[0]).wait()
    pltpu.make_async_copy(ones_v, deg_sh.at[sidx[1]], ssem[1]).wait()
    plsc.subcore_barrier()

    pltpu.sync_copy(deg_sh.at[pl.ds(zb, RB)], rb_v)
    pltpu.sync_copy(rb_v, out_hbm.at[cid, pl.ds(zb, RB)])


@functools.partial(
    pl.kernel,
    out_type=(
        jax.ShapeDtypeStruct((NC, NP, DH), jnp.float32),
        jax.ShapeDtypeStruct((NC, NP), jnp.float32),
    ),
    mesh=_mesh,
    scratch_types=(
        [pltpu.VMEM((K,), jnp.int32)] * 2
        + [pltpu.VMEM((K,), jnp.int32)] * 2
        + [pltpu.VMEM((K,), jnp.int32)] * 2
        + [pltpu.VMEM((K,), jnp.int32)] * 2
        + [pltpu.VMEM((K,), jnp.int32)] * 2
        + [pltpu.VMEM((K, DH), jnp.float32)] * 2
        + [pltpu.VMEM((K,), jnp.float32)] * 2
        + [
            pltpu.VMEM((RB, DH), jnp.float32),
            pltpu.VMEM((RB,), jnp.float32),
            pltpu.VMEM_SHARED((NP, DH), jnp.float32),
            pltpu.VMEM_SHARED((NP,), jnp.float32),
        ]
        + [pltpu.SemaphoreType.DMA] * 12
    ),
    compiler_params=pltpu.CompilerParams(use_tc_tiling_on_sc=False),
)
def _sc_edges(
    ei_hbm, sflat_hbm, dis_hbm,
    acc_out, csum_out,
    sr0, sr1, dsl0, dsl1, gx0, gx1, sd0, sd1, cs0, cs1, rw0, rw1, dv0, dv1,
    rbr_v, rbv_v, acc_sh, csum_sh,
    lis0, lis1, lid0, lid1, lg0, lg1, ld0, ld1, lr0, lr1, lc0, lc1,
):
    cid = lax.axis_index("c")
    sid = lax.axis_index("s")
    zb = pl.multiple_of(sid * RB, 8)
    goff = cid * NP

    srci = (sr0, sr1)
    dsti = (dsl0, dsl1)
    gidx = (gx0, gx1)
    sdst = (sd0, sd1)
    csrc = (cs0, cs1)
    rows = (rw0, rw1)
    disv = (dv0, dv1)
    isems = (lis0, lis1)
    isemd = (lid0, lid1)
    gsem = (lg0, lg1)
    dsem = (ld0, ld1)
    ssem = (lr0, lr1)
    csem = (lc0, lc1)

    def zfill(r, carry):
        for j in range(DH // L):
            rbr_v[r, pl.ds(j * L, L)] = _z16()
        return carry

    lax.fori_loop(0, RB, zfill, 0)
    for i in range(RB // L):
        rbv_v[pl.ds(L * i, L)] = _z16()
    pltpu.sync_copy(rbr_v, acc_sh.at[pl.ds(zb, RB)])
    pltpu.sync_copy(rbv_v, csum_sh.at[pl.ds(zb, RB)])
    plsc.subcore_barrier()

    rbase = pl.multiple_of(sid * EPT, 8)
    NCH = EPT // K
    NPAIR = NCH // 2

    def coff(c):
        return pl.multiple_of(rbase + c * K, 8)

    for b in range(2):
        pltpu.async_copy(ei_hbm.at[0, pl.ds(coff(b), K)], srci[b], isems[b])
        pltpu.async_copy(ei_hbm.at[1, pl.ds(coff(b), K)], dsti[b], isemd[b])

    def body(i2, carry):
        for b in range(2):
            c = i2 * 2 + b
            pltpu.make_async_copy(
                ei_hbm.at[0, pl.ds(coff(c), K)], srci[b], isems[b]
            ).wait()
            pltpu.make_async_copy(
                ei_hbm.at[1, pl.ds(coff(c), K)], dsti[b], isemd[b]
            ).wait()

            @pl.when(i2 > 0)
            def _():
                pltpu.make_async_copy(
                    rows[b], acc_sh.at[sdst[b]], ssem[b]
                ).wait()
                pltpu.make_async_copy(
                    disv[b], csum_sh.at[csrc[b]], csem[b]
                ).wait()

            for j in range(K // L):
                sl = pl.ds(j * L, L)
                sv = srci[b][sl]
                gidx[b][sl] = sv + goff
                csrc[b][sl] = sv
                sdst[b][sl] = dsti[b][sl]

            @pl.when(i2 < NPAIR - 1)
            def _():
                pltpu.async_copy(
                    ei_hbm.at[0, pl.ds(coff(c + 2), K)], srci[b], isems[b]
                )
                pltpu.async_copy(
                    ei_hbm.at[1, pl.ds(coff(c + 2), K)], dsti[b], isemd[b]
                )

            pltpu.async_copy(sflat_hbm.at[gidx[b]], rows[b], gsem[b])
            pltpu.async_copy(dis_hbm.at[sdst[b]], disv[b], dsem[b])

        for b in range(2):
            pltpu.make_async_copy(sflat_hbm.at[gidx[b]], rows[b], gsem[b]).wait()
            pltpu.async_copy(rows[b], acc_sh.at[sdst[b]], ssem[b], add=True)
            pltpu.make_async_copy(dis_hbm.at[sdst[b]], disv[b], dsem[b]).wait()
            pltpu.async_copy(disv[b], csum_sh.at[csrc[b]], csem[b], add=True)
        return carry

    lax.fori_loop(0, NPAIR, body, 0)
    for b in range(2):
        pltpu.make_async_copy(rows[b], acc_sh.at[sdst[b]], ssem[b]).wait()
        pltpu.make_async_copy(disv[b], csum_sh.at[csrc[b]], csem[b]).wait()
    plsc.subcore_barrier()

    pltpu.sync_copy(acc_sh.at[pl.ds(zb, RB)], rbr_v)
    pltpu.sync_copy(rbr_v, acc_out.at[cid, pl.ds(zb, RB)])
    pltpu.sync_copy(csum_sh.at[pl.ds(zb, RB)], rbv_v)
    pltpu.sync_copy(rbv_v, csum_out.at[cid, pl.ds(zb, RB)])


@functools.partial(
    pl.kernel,
    out_type=jax.ShapeDtypeStruct((NC, NS, DH), jnp.float32),
    mesh=_mesh,
    scratch_types=[
        pltpu.VMEM((RB, DH), jnp.float32),
        pltpu.VMEM((RB, DH), jnp.float32),
        pltpu.VMEM((RB,), jnp.float32),
        pltpu.VMEM((RB,), jnp.float32),
        pltpu.VMEM((DH,), jnp.float32),
        pltpu.VMEM((DH,), jnp.float32),
        pltpu.SemaphoreType.DMA,
        pltpu.SemaphoreType.DMA,
        pltpu.SemaphoreType.DMA,
        pltpu.SemaphoreType.DMA,
    ],
    compiler_params=pltpu.CompilerParams(use_tc_tiling_on_sc=False),
)
def _sc_reduce(
    acc_hbm, csum_hbm, sflat_hbm, dis_hbm, b1_hbm,
    gp_out,
    av_v, sv_v, cv_v, dv_v, b1h_v, gv_v, la, lb, lc, ld,
):
    cid = lax.axis_index("c")
    sid = lax.axis_index("s")
    zb = pl.multiple_of(sid * RB, 8)
    spos = pl.multiple_of(cid * NP + zb, 8)
    bpos = pl.multiple_of(cid * DH, 8)

    pltpu.async_copy(acc_hbm.at[cid, pl.ds(zb, RB)], av_v, la)
    pltpu.async_copy(sflat_hbm.at[pl.ds(spos, RB)], sv_v, lb)
    pltpu.async_copy(csum_hbm.at[cid, pl.ds(zb, RB)], cv_v, lc)
    pltpu.async_copy(dis_hbm.at[pl.ds(zb, RB)], dv_v, ld)
    pltpu.sync_copy(b1_hbm.at[pl.ds(bpos, DH)], b1h_v)
    pltpu.make_async_copy(acc_hbm.at[cid, pl.ds(zb, RB)], av_v, la).wait()
    pltpu.make_async_copy(sflat_hbm.at[pl.ds(spos, RB)], sv_v, lb).wait()
    pltpu.make_async_copy(csum_hbm.at[cid, pl.ds(zb, RB)], cv_v, lc).wait()
    pltpu.make_async_copy(dis_hbm.at[pl.ds(zb, RB)], dv_v, ld).wait()

    limit = jnp.where(sid == NS - 1, RB - (NP - N), RB)

    def _splat(vec, lane):
        return lax.gather(
            vec,
            jnp.full((L, 1), lane, jnp.int32),
            lax.GatherDimensionNumbers(
                offset_dims=(),
                collapsed_slice_dims=(0,),
                start_index_map=(0,),
            ),
            (1,),
            mode=lax.GatherScatterMode.PROMISE_IN_BOUNDS,
        )

    def red_body(gi, g):
        base = gi * L
        dis16 = dv_v[pl.ds(base, L)]
        cs16 = cv_v[pl.ds(base, L)]
        rowid = lax.iota(jnp.int32, L) + jnp.full((L,), base, jnp.int32)
        mask = jnp.where(
            rowid < jnp.full((L,), limit, jnp.int32),
            jnp.full((L,), 1.0, jnp.float32),
            _z16(),
        )
        c16 = dis16 * (dis16 + cs16) * mask
        out = list(g)
        for rr in range(L):
            dsp = _splat(dis16, rr)
            csp = _splat(c16, rr)
            r = base + rr
            for j in range(DH // L):
                sl = pl.ds(j * L, L)
                hv = jnp.maximum(
                    dsp * (av_v[r, sl] + sv_v[r, sl]) + b1h_v[sl], 0.0
                )
                out[j] = out[j] + csp * hv
        return tuple(out)

    g = lax.fori_loop(
        0, RB // L, red_body, tuple(_z16() for _ in range(DH // L))
    )
    for j in range(DH // L):
        gv_v[pl.ds(j * L, L)] = g[j]
    pltpu.sync_copy(gv_v, gp_out.at[cid, sid])


BLK = 1000


def _tc_scale_body(x_ref, w_ref, d0_ref, d1_ref, sp_ref, dis_ref):
    xw = jnp.dot(x_ref[...], w_ref[...], preferred_element_type=jnp.float32)
    deg = d0_ref[0] + d1_ref[0] + 1.0
    dis = lax.rsqrt(deg)
    s = xw * dis
    sp_ref[0] = s[:, :DH]
    sp_ref[1] = s[:, DH:]
    dis_ref[...] = dis


def _tc_scale(x, W1, degp):
    degp3 = degp.reshape(NC, NP, 1)
    return pl.pallas_call(
        _tc_scale_body,
        grid=(N // BLK,),
        in_specs=[
            pl.BlockSpec((BLK, D), lambda i: (i, 0)),
            pl.BlockSpec((D, D), lambda i: (0, 0)),
            pl.BlockSpec((1, BLK, 1), lambda i: (0, i, 0)),
            pl.BlockSpec((1, BLK, 1), lambda i: (1, i, 0)),
        ],
        out_specs=[
            pl.BlockSpec((NC, BLK, DH), lambda i: (0, i, 0)),
            pl.BlockSpec((BLK, 1), lambda i: (i, 0)),
        ],
        out_shape=[
            jax.ShapeDtypeStruct((NC, NP, DH), jnp.float32),
            jax.ShapeDtypeStruct((NP, 1), jnp.float32),
        ],
        compiler_params=pltpu.CompilerParams(
            dimension_semantics=("arbitrary",),
        ),
    )(x, W1, degp3, degp3)


def _tc_final_body(gp_ref, wmu_ref, bmu_ref, wlv_ref, blv_ref, mu_ref, lv_ref):
    g = jnp.concatenate(
        [
            jnp.sum(gp_ref[0], axis=0, keepdims=True),
            jnp.sum(gp_ref[1], axis=0, keepdims=True),
        ],
        axis=1,
    )
    fn = jnp.float32(N)
    mu_ref[...] = (
        jnp.dot(g, wmu_ref[...], preferred_element_type=jnp.float32)
        + fn * bmu_ref[...]
    )
    lv_ref[...] = (
        jnp.dot(g, wlv_ref[...], preferred_element_type=jnp.float32)
        + fn * blv_ref[...]
    )


def _tc_final(gp, W_mu, b_mu, W_lv, b_lv):
    return pl.pallas_call(
        _tc_final_body,
        out_shape=[
            jax.ShapeDtypeStruct((1, DO), jnp.float32),
            jax.ShapeDtypeStruct((1, DO), jnp.float32),
        ],
    )(gp, W_mu, b_mu.reshape(1, DO), W_lv, b_lv.reshape(1, DO))


def kernel(x, edge_index, W1, b1, W_mu, b_mu, W_lv, b_lv):
    ei = edge_index.astype(jnp.int32)
    degp = _sc_degree(ei)
    s_pair, dis = _tc_scale(x, W1, degp)
    sflat = s_pair.reshape(NC * NP, DH)
    disf = dis.reshape(NP)
    accp, csump = _sc_edges(ei, sflat, disf)
    gp = _sc_reduce(accp, csump, sflat, disf, b1)
    mu, lv = _tc_final(gp, W_mu, b_mu, W_lv, b_lv)
    return (mu, lv)
